# initial kernel scaffold (unmeasured)
import jax
import jax.numpy as jnp
from jax import lax
from jax.experimental import pallas as pl
from jax.experimental.pallas import tpu as pltpu

N_DEV = 32


def kernel(x, w_mat):
    m_per, k = x.shape
    _, n = w_mat.shape
    n_per = n // N_DEV

    def body(x_ref, w_ref, out_ref,
             y_src, recv_buf, amax_src, amax_recv,
             send_sems, recv_sems, am_send_sems, am_recv_sems):
        my = lax.axis_index("i")

        y = jnp.dot(x_ref[:, :], w_ref[:, :],
                    preferred_element_type=jnp.float32)
        local_amax = jnp.max(jnp.abs(y))
        amax_src[0, :] = jnp.full((128,), local_amax, dtype=jnp.float32)

        yb = y.astype(jnp.bfloat16)
        for j in range(N_DEV):
            y_src[j, :, :] = yb[:, j * n_per:(j + 1) * n_per]

        recv_buf[my, :, :] = y_src[my, :, :]
        amax_recv[my, :] = jnp.full((128,), local_amax, dtype=jnp.float32)

        for j in range(N_DEV):
            @pl.when(j != my)
            def _(j=j):
                pltpu.make_async_remote_copy(
                    src_ref=y_src.at[j],
                    dst_ref=recv_buf.at[my],
                    send_sem=send_sems.at[j],
                    recv_sem=recv_sems.at[my],
                    device_id=(j,),
                    device_id_type=pl.DeviceIdType.MESH,
                ).start()
                pltpu.make_async_remote_copy(
                    src_ref=amax_src,
                    dst_ref=amax_recv.at[pl.ds(my, 1), :],
                    send_sem=am_send_sems.at[j],
                    recv_sem=am_recv_sems.at[my],
                    device_id=(j,),
                    device_id_type=pl.DeviceIdType.MESH,
                ).start()

        for s in range(N_DEV):
            @pl.when(s != my)
            def _(s=s):
                pltpu.make_async_remote_copy(
                    src_ref=amax_src,
                    dst_ref=amax_recv.at[pl.ds(s, 1), :],
                    send_sem=am_send_sems.at[s],
                    recv_sem=am_recv_sems.at[s],
                    device_id=(my,),
                    device_id_type=pl.DeviceIdType.MESH,
                ).wait_recv()
        g_amax = jnp.max(amax_recv[:, :])
        inv_scale = 127.0 / g_amax
        scale = g_amax / 127.0

        for s in range(N_DEV):
            @pl.when(s != my)
            def _(s=s):
                pltpu.make_async_remote_copy(
                    src_ref=y_src.at[s],
                    dst_ref=recv_buf.at[s],
                    send_sem=send_sems.at[s],
                    recv_sem=recv_sems.at[s],
                    device_id=(my,),
                    device_id_type=pl.DeviceIdType.MESH,
                ).wait_recv()
        for s in range(N_DEV):
            yf = recv_buf[s, :, :].astype(jnp.float32)
            q = jnp.clip(jnp.round(yf * inv_scale), -127.0, 127.0)
            out_ref[s * m_per:(s + 1) * m_per, :] = q * scale

        for j in range(N_DEV):
            @pl.when(j != my)
            def _(j=j):
                pltpu.make_async_remote_copy(
                    src_ref=y_src.at[j],
                    dst_ref=recv_buf.at[my],
                    send_sem=send_sems.at[j],
                    recv_sem=recv_sems.at[my],
                    device_id=(j,),
                    device_id_type=pl.DeviceIdType.MESH,
                ).wait_send()
                pltpu.make_async_remote_copy(
                    src_ref=amax_src,
                    dst_ref=amax_recv.at[pl.ds(my, 1), :],
                    send_sem=am_send_sems.at[j],
                    recv_sem=am_recv_sems.at[my],
                    device_id=(j,),
                    device_id_type=pl.DeviceIdType.MESH,
                ).wait_send()

    return pl.pallas_call(
        body,
        out_shape=jax.ShapeDtypeStruct((N_DEV * m_per, n_per), jnp.float32),
        in_specs=[
            pl.BlockSpec(memory_space=pltpu.VMEM),
            pl.BlockSpec(memory_space=pltpu.VMEM),
        ],
        out_specs=pl.BlockSpec(memory_space=pltpu.VMEM),
        scratch_shapes=[
            pltpu.VMEM((N_DEV, m_per, n_per), jnp.bfloat16),
            pltpu.VMEM((N_DEV, m_per, n_per), jnp.bfloat16),
            pltpu.VMEM((1, 128), jnp.float32),
            pltpu.VMEM((N_DEV, 128), jnp.float32),
            pltpu.SemaphoreType.DMA((N_DEV,)),
            pltpu.SemaphoreType.DMA((N_DEV,)),
            pltpu.SemaphoreType.DMA((N_DEV,)),
            pltpu.SemaphoreType.DMA((N_DEV,)),
        ],
    )(x, w_mat)


# baseline (device time: 43929 ns/iter reference)
import jax
import jax.numpy as jnp
from jax import lax
from jax.experimental import pallas as pl
from jax.experimental.pallas import tpu as pltpu

N_DEV = 32


def kernel(x, w_mat):
    m_per, k = x.shape
    _, n = w_mat.shape
    n_per = n // N_DEV

    def body(x_ref, w_ref, out_ref,
             y_src, recv_buf, amax_src, amax_recv,
             send_sems, recv_sems, am_send_sems, am_recv_sems):
        my = lax.axis_index("i")

        y = jnp.dot(x_ref[:, :].astype(jnp.bfloat16),
                    w_ref[:, :].astype(jnp.bfloat16),
                    preferred_element_type=jnp.float32)
        local_amax = jnp.max(jnp.abs(y))
        amax_src[0, :] = jnp.full((128,), local_amax, dtype=jnp.float32)

        yb = y.astype(jnp.bfloat16)
        for j in range(N_DEV):
            y_src[j, :, :] = yb[:, j * n_per:(j + 1) * n_per]

        recv_buf[my, :, :] = y_src[my, :, :]
        amax_recv[my, :] = jnp.full((128,), local_amax, dtype=jnp.float32)

        for j in range(N_DEV):
            @pl.when(j != my)
            def _(j=j):
                pltpu.make_async_remote_copy(
                    src_ref=y_src.at[j],
                    dst_ref=recv_buf.at[my],
                    send_sem=send_sems.at[j],
                    recv_sem=recv_sems.at[my],
                    device_id=(j,),
                    device_id_type=pl.DeviceIdType.MESH,
                ).start()
                pltpu.make_async_remote_copy(
                    src_ref=amax_src,
                    dst_ref=amax_recv.at[pl.ds(my, 1), :],
                    send_sem=am_send_sems.at[j],
                    recv_sem=am_recv_sems.at[my],
                    device_id=(j,),
                    device_id_type=pl.DeviceIdType.MESH,
                ).start()

        for s in range(N_DEV):
            @pl.when(s != my)
            def _(s=s):
                pltpu.make_async_remote_copy(
                    src_ref=amax_src,
                    dst_ref=amax_recv.at[pl.ds(s, 1), :],
                    send_sem=am_send_sems.at[s],
                    recv_sem=am_recv_sems.at[s],
                    device_id=(my,),
                    device_id_type=pl.DeviceIdType.MESH,
                ).wait_recv()
        g_amax = jnp.max(amax_recv[:, :])
        inv_scale = 127.0 / g_amax
        scale = g_amax / 127.0

        for s in range(N_DEV):
            @pl.when(s != my)
            def _(s=s):
                pltpu.make_async_remote_copy(
                    src_ref=y_src.at[s],
                    dst_ref=recv_buf.at[s],
                    send_sem=send_sems.at[s],
                    recv_sem=recv_sems.at[s],
                    device_id=(my,),
                    device_id_type=pl.DeviceIdType.MESH,
                ).wait_recv()
        for s in range(N_DEV):
            yf = recv_buf[s, :, :].astype(jnp.float32)
            q = jnp.clip(jnp.round(yf * inv_scale), -127.0, 127.0)
            out_ref[s * m_per:(s + 1) * m_per, :] = q * scale

        for j in range(N_DEV):
            @pl.when(j != my)
            def _(j=j):
                pltpu.make_async_remote_copy(
                    src_ref=y_src.at[j],
                    dst_ref=recv_buf.at[my],
                    send_sem=send_sems.at[j],
                    recv_sem=recv_sems.at[my],
                    device_id=(j,),
                    device_id_type=pl.DeviceIdType.MESH,
                ).wait_send()
                pltpu.make_async_remote_copy(
                    src_ref=amax_src,
                    dst_ref=amax_recv.at[pl.ds(my, 1), :],
                    send_sem=am_send_sems.at[j],
                    recv_sem=am_recv_sems.at[my],
                    device_id=(j,),
                    device_id_type=pl.DeviceIdType.MESH,
                ).wait_send()

    return pl.pallas_call(
        body,
        out_shape=jax.ShapeDtypeStruct((N_DEV * m_per, n_per), jnp.float32),
        in_specs=[
            pl.BlockSpec(memory_space=pltpu.VMEM),
            pl.BlockSpec(memory_space=pltpu.VMEM),
        ],
        out_specs=pl.BlockSpec(memory_space=pltpu.VMEM),
        scratch_shapes=[
            pltpu.VMEM((N_DEV, m_per, n_per), jnp.bfloat16),
            pltpu.VMEM((N_DEV, m_per, n_per), jnp.bfloat16),
            pltpu.VMEM((1, 128), jnp.float32),
            pltpu.VMEM((N_DEV, 128), jnp.float32),
            pltpu.SemaphoreType.DMA((N_DEV,)),
            pltpu.SemaphoreType.DMA((N_DEV,)),
            pltpu.SemaphoreType.DMA((N_DEV,)),
            pltpu.SemaphoreType.DMA((N_DEV,)),
        ],
        compiler_params=pltpu.CompilerParams(
            vmem_limit_bytes=100 * 1024 * 1024,
        ),
    )(x, w_mat)


# device time: 41321 ns/iter; 1.0631x vs baseline; 1.0631x over previous
import jax
import jax.numpy as jnp
from jax import lax
from jax.experimental import pallas as pl
from jax.experimental.pallas import tpu as pltpu

N_DEV = 32
N_CHUNKS = 8
DEV_PER_CHUNK = N_DEV // N_CHUNKS


def kernel(x, w_mat):
    m_per, k = x.shape
    _, n = w_mat.shape
    n_per = n // N_DEV
    n_ck = n // N_CHUNKS

    def body(x_ref, w_hbm, out_ref,
             w_vmem, y_src, recv2d, amax_src, amax_recv,
             w_sems, send_sems, recv_sems, am_send_sems, am_recv_sems):
        my = lax.axis_index("i")

        with jax.named_scope("w_dma_issue"):
            for c in range(N_CHUNKS):
                pltpu.make_async_copy(
                    w_hbm.at[:, c * n_ck:(c + 1) * n_ck],
                    w_vmem.at[c],
                    w_sems.at[c],
                ).start()

        with jax.named_scope("x_cast"):
            xb = x_ref[:, :].astype(jnp.bfloat16)

        amax = jnp.float32(0.0)
        for c in range(N_CHUNKS):
            with jax.named_scope(f"w_wait#{c}"):
                pltpu.make_async_copy(
                    w_hbm.at[:, c * n_ck:(c + 1) * n_ck],
                    w_vmem.at[c],
                    w_sems.at[c],
                ).wait()
            with jax.named_scope(f"mm#{c}"):
                wb = w_vmem[c].astype(jnp.bfloat16)
                yc = jnp.dot(xb, wb,
                             preferred_element_type=jnp.float32)
                amax = jnp.maximum(amax, jnp.max(jnp.abs(yc)))
                ycb = yc.astype(jnp.bfloat16)
            with jax.named_scope(f"store_send#{c}"):
                for t in range(DEV_PER_CHUNK):
                    j = c * DEV_PER_CHUNK + t
                    y_src[j, :, :] = ycb[:, t * n_per:(t + 1) * n_per]

                    @pl.when(j == my)
                    def _(j=j):
                        recv2d[pl.ds(j * m_per, m_per), :] = y_src[j, :, :]

                    @pl.when(j != my)
                    def _(j=j):
                        pltpu.make_async_remote_copy(
                            src_ref=y_src.at[j],
                            dst_ref=recv2d.at[pl.ds(my * m_per, m_per), :],
                            send_sem=send_sems.at[j],
                            recv_sem=recv_sems.at[my],
                            device_id=(j,),
                            device_id_type=pl.DeviceIdType.MESH,
                        ).start()

        with jax.named_scope("amax_send"):
            amax_src[0, :] = jnp.full((128,), amax, dtype=jnp.float32)
            amax_recv[my, :] = jnp.full((128,), amax, dtype=jnp.float32)
            for j in range(N_DEV):
                @pl.when(j != my)
                def _(j=j):
                    pltpu.make_async_remote_copy(
                        src_ref=amax_src,
                        dst_ref=amax_recv.at[pl.ds(my, 1), :],
                        send_sem=am_send_sems.at[j],
                        recv_sem=am_recv_sems.at[my],
                        device_id=(j,),
                        device_id_type=pl.DeviceIdType.MESH,
                    ).start()

        with jax.named_scope("amax_wait"):
            for s in range(N_DEV):
                @pl.when(s != my)
                def _(s=s):
                    pltpu.make_async_remote_copy(
                        src_ref=amax_src,
                        dst_ref=amax_recv.at[pl.ds(s, 1), :],
                        send_sem=am_send_sems.at[s],
                        recv_sem=am_recv_sems.at[s],
                        device_id=(my,),
                        device_id_type=pl.DeviceIdType.MESH,
                    ).wait_recv()
            g_amax = jnp.max(amax_recv[:, :])
        inv_scale = 127.0 / g_amax
        scale = g_amax / 127.0

        with jax.named_scope("data_wait"):
            for s in range(N_DEV):
                @pl.when(s != my)
                def _(s=s):
                    pltpu.make_async_remote_copy(
                        src_ref=y_src.at[s],
                        dst_ref=recv2d.at[pl.ds(s * m_per, m_per), :],
                        send_sem=send_sems.at[s],
                        recv_sem=recv_sems.at[s],
                        device_id=(my,),
                        device_id_type=pl.DeviceIdType.MESH,
                    ).wait_recv()

        with jax.named_scope("quant"):
            yf = recv2d[:, :].astype(jnp.float32)
            q = jnp.clip(jnp.round(yf * inv_scale), -127.0, 127.0)
            out_ref[:, :] = q * scale

        with jax.named_scope("drain"):
            for j in range(N_DEV):
                @pl.when(j != my)
                def _(j=j):
                    pltpu.make_async_remote_copy(
                        src_ref=y_src.at[j],
                        dst_ref=recv2d.at[pl.ds(my * m_per, m_per), :],
                        send_sem=send_sems.at[j],
                        recv_sem=recv_sems.at[my],
                        device_id=(j,),
                        device_id_type=pl.DeviceIdType.MESH,
                    ).wait_send()
                    pltpu.make_async_remote_copy(
                        src_ref=amax_src,
                        dst_ref=amax_recv.at[pl.ds(my, 1), :],
                        send_sem=am_send_sems.at[j],
                        recv_sem=am_recv_sems.at[my],
                        device_id=(j,),
                        device_id_type=pl.DeviceIdType.MESH,
                    ).wait_send()

    return pl.pallas_call(
        body,
        out_shape=jax.ShapeDtypeStruct((N_DEV * m_per, n_per), jnp.float32),
        in_specs=[
            pl.BlockSpec(memory_space=pltpu.VMEM),
            pl.BlockSpec(memory_space=pltpu.MemorySpace.HBM),
        ],
        out_specs=pl.BlockSpec(memory_space=pltpu.VMEM),
        scratch_shapes=[
            pltpu.VMEM((N_CHUNKS, k, n // N_CHUNKS), jnp.float32),
            pltpu.VMEM((N_DEV, m_per, n_per), jnp.bfloat16),
            pltpu.VMEM((N_DEV * m_per, n_per), jnp.bfloat16),
            pltpu.VMEM((1, 128), jnp.float32),
            pltpu.VMEM((N_DEV, 128), jnp.float32),
            pltpu.SemaphoreType.DMA((N_CHUNKS,)),
            pltpu.SemaphoreType.DMA((N_DEV,)),
            pltpu.SemaphoreType.DMA((N_DEV,)),
            pltpu.SemaphoreType.DMA((N_DEV,)),
            pltpu.SemaphoreType.DMA((N_DEV,)),
        ],
        compiler_params=pltpu.CompilerParams(
            vmem_limit_bytes=100 * 1024 * 1024,
        ),
    )(x, w_mat)


# device time: 20743 ns/iter; 2.1178x vs baseline; 1.9920x over previous
import os

import jax
import jax.numpy as jnp
from jax import lax
from jax.experimental import pallas as pl
from jax.experimental.pallas import tpu as pltpu

_ABLATE = os.environ.get("ABLATE", "")
_DO_DATA = _ABLATE != "nocomm"
_DO_AMAX = _ABLATE not in ("nocomm", "noamax")

N_DEV = 32
N_CHUNKS = 8
DEV_PER_CHUNK = N_DEV // N_CHUNKS


def kernel(x, w_mat):
    m_per, k = x.shape
    _, n = w_mat.shape
    n_per = n // N_DEV
    n_ck = n // N_CHUNKS

    def body(x_ref, w_hbm, out_ref,
             w_vmem, y_src, recv2d, amax_src, amax_recv,
             w_sems, send_sems, recv_sems, am_send_sems, am_recv_sems):
        my = lax.axis_index("i")

        with jax.named_scope("w_dma_issue"):
            for c in range(N_CHUNKS):
                pltpu.make_async_copy(
                    w_hbm.at[:, c * n_ck:(c + 1) * n_ck],
                    w_vmem.at[c],
                    w_sems.at[c],
                ).start()

        with jax.named_scope("x_cast"):
            xb = x_ref[:, :].astype(jnp.bfloat16)

        amax = jnp.float32(0.0)
        for c in range(N_CHUNKS):
            with jax.named_scope(f"w_wait#{c}"):
                pltpu.make_async_copy(
                    w_hbm.at[:, c * n_ck:(c + 1) * n_ck],
                    w_vmem.at[c],
                    w_sems.at[c],
                ).wait()
            with jax.named_scope(f"mm#{c}"):
                wb = w_vmem[c].astype(jnp.bfloat16)
                yc = jnp.dot(xb, wb,
                             preferred_element_type=jnp.float32)
                amax = jnp.maximum(amax, jnp.max(jnp.abs(yc)))
                ycb = yc.astype(jnp.bfloat16)
            with jax.named_scope(f"store_send#{c}"):
                for t in range(DEV_PER_CHUNK):
                    j = c * DEV_PER_CHUNK + t
                    y_src[j, :, :] = ycb[:, t * n_per:(t + 1) * n_per]

                    @pl.when(j == my)
                    def _(j=j):
                        recv2d[pl.ds(j * m_per, m_per), :] = y_src[j, :, :]

                    if _DO_DATA:
                        @pl.when(j != my)
                        def _(j=j):
                            pltpu.make_async_remote_copy(
                                src_ref=y_src.at[j],
                                dst_ref=recv2d.at[pl.ds(my * m_per, m_per), :],
                                send_sem=send_sems.at[j],
                                recv_sem=recv_sems.at[my],
                                device_id=(j,),
                                device_id_type=pl.DeviceIdType.MESH,
                            ).start()

        with jax.named_scope("amax_send"):
            amax_src[0, :] = jnp.full((128,), amax, dtype=jnp.float32)
            amax_recv[my, :] = jnp.full((128,), amax, dtype=jnp.float32)
            if _DO_AMAX:
                for j in range(N_DEV):
                    @pl.when(j != my)
                    def _(j=j):
                        pltpu.make_async_remote_copy(
                            src_ref=amax_src,
                            dst_ref=amax_recv.at[pl.ds(my, 1), :],
                            send_sem=am_send_sems.at[j],
                            recv_sem=am_recv_sems.at[my],
                            device_id=(j,),
                            device_id_type=pl.DeviceIdType.MESH,
                        ).start()

        with jax.named_scope("amax_wait"):
            if _DO_AMAX:
                for s in range(N_DEV):
                    @pl.when(s != my)
                    def _(s=s):
                        pltpu.make_async_remote_copy(
                            src_ref=amax_src,
                            dst_ref=amax_recv.at[pl.ds(s, 1), :],
                            send_sem=am_send_sems.at[s],
                            recv_sem=am_recv_sems.at[s],
                            device_id=(my,),
                            device_id_type=pl.DeviceIdType.MESH,
                        ).wait_recv()
                g_amax = jnp.max(amax_recv[:, :])
            else:
                g_amax = amax
        inv_scale = 127.0 / g_amax
        scale = g_amax / 127.0

        with jax.named_scope("data_wait"):
            for s in range(N_DEV) if _DO_DATA else ():
                @pl.when(s != my)
                def _(s=s):
                    pltpu.make_async_remote_copy(
                        src_ref=y_src.at[s],
                        dst_ref=recv2d.at[pl.ds(s * m_per, m_per), :],
                        send_sem=send_sems.at[s],
                        recv_sem=recv_sems.at[s],
                        device_id=(my,),
                        device_id_type=pl.DeviceIdType.MESH,
                    ).wait_recv()

        with jax.named_scope("quant"):
            yf = recv2d[:, :].astype(jnp.float32)
            q = jnp.clip(jnp.round(yf * inv_scale), -127.0, 127.0)
            out_ref[:, :] = q * scale

        with jax.named_scope("drain"):
            for j in range(N_DEV):
                if _DO_DATA:
                    @pl.when(j != my)
                    def _(j=j):
                        pltpu.make_async_remote_copy(
                            src_ref=y_src.at[j],
                            dst_ref=recv2d.at[pl.ds(my * m_per, m_per), :],
                            send_sem=send_sems.at[j],
                            recv_sem=recv_sems.at[my],
                            device_id=(j,),
                            device_id_type=pl.DeviceIdType.MESH,
                        ).wait_send()
                if _DO_AMAX:
                    @pl.when(j != my)
                    def _(j=j):
                        pltpu.make_async_remote_copy(
                            src_ref=amax_src,
                            dst_ref=amax_recv.at[pl.ds(my, 1), :],
                            send_sem=am_send_sems.at[j],
                            recv_sem=am_recv_sems.at[my],
                            device_id=(j,),
                            device_id_type=pl.DeviceIdType.MESH,
                        ).wait_send()

    return pl.pallas_call(
        body,
        out_shape=jax.ShapeDtypeStruct((N_DEV * m_per, n_per), jnp.float32),
        in_specs=[
            pl.BlockSpec(memory_space=pltpu.VMEM),
            pl.BlockSpec(memory_space=pltpu.MemorySpace.HBM),
        ],
        out_specs=pl.BlockSpec(memory_space=pltpu.VMEM),
        scratch_shapes=[
            pltpu.VMEM((N_CHUNKS, k, n // N_CHUNKS), jnp.float32),
            pltpu.VMEM((N_DEV, m_per, n_per), jnp.bfloat16),
            pltpu.VMEM((N_DEV * m_per, n_per), jnp.bfloat16),
            pltpu.VMEM((1, 128), jnp.float32),
            pltpu.VMEM((N_DEV, 128), jnp.float32),
            pltpu.SemaphoreType.DMA((N_CHUNKS,)),
            pltpu.SemaphoreType.DMA((N_DEV,)),
            pltpu.SemaphoreType.DMA((N_DEV,)),
            pltpu.SemaphoreType.DMA((N_DEV,)),
            pltpu.SemaphoreType.DMA((N_DEV,)),
        ],
        compiler_params=pltpu.CompilerParams(
            vmem_limit_bytes=100 * 1024 * 1024,
        ),
    )(x, w_mat)


# device time: 18671 ns/iter; 2.3528x vs baseline; 1.1110x over previous
import os

import jax
import jax.numpy as jnp
from jax import lax
from jax.experimental import pallas as pl
from jax.experimental.pallas import tpu as pltpu

_ABLATE = os.environ.get("ABLATE", "")
_DO_DATA = _ABLATE not in ("nocomm", "dmaonly")
_DO_AMAX = _ABLATE not in ("nocomm", "noamax", "dmaonly")
_DO_MATH = _ABLATE != "dmaonly"

N_DEV = 32
N_CHUNKS = 8
DEV_PER_CHUNK = N_DEV // N_CHUNKS


def kernel(x, w_mat):
    m_per, k = x.shape
    _, n = w_mat.shape
    n_per = n // N_DEV
    n_ck = n // N_CHUNKS

    def body(x_ref, w_hbm, out_ref,
             w_vmem, y_src, recv2d, amax_src, amax_recv,
             w_sems, send_sems, recv_sems, am_send_sems, am_recv_sems):
        my = lax.axis_index("i")

        with jax.named_scope("w_dma_issue"):
            for c in range(N_CHUNKS):
                pltpu.make_async_copy(
                    w_hbm.at[:, c * n_ck:(c + 1) * n_ck],
                    w_vmem.at[c],
                    w_sems.at[c],
                ).start()

        with jax.named_scope("x_cast"):
            xb = x_ref[:, :].astype(jnp.bfloat16)

        amax = jnp.float32(0.0)
        for c in range(N_CHUNKS):
            with jax.named_scope(f"w_wait#{c}"):
                pltpu.make_async_copy(
                    w_hbm.at[:, c * n_ck:(c + 1) * n_ck],
                    w_vmem.at[c],
                    w_sems.at[c],
                ).wait()
            if _DO_MATH:
                with jax.named_scope(f"mm#{c}"):
                    wb = w_vmem[c].astype(jnp.bfloat16)
                    yc = jnp.dot(xb, wb,
                                 preferred_element_type=jnp.float32)
                    amax = jnp.maximum(amax, jnp.max(jnp.abs(yc)))
                    ycb = yc.astype(jnp.bfloat16)
            else:
                amax = jnp.maximum(amax, jnp.max(jnp.abs(w_vmem[c, 0, :])))
                ycb = None
            with jax.named_scope(f"store_send#{c}"):
                for t in range(DEV_PER_CHUNK):
                    j = c * DEV_PER_CHUNK + t
                    if _DO_MATH:
                        y_src[j, :, :] = ycb[:, t * n_per:(t + 1) * n_per]

                    @pl.when(j == my)
                    def _(j=j):
                        recv2d[pl.ds(j * m_per, m_per), :] = y_src[j, :, :]

                    if _DO_DATA:
                        @pl.when(j != my)
                        def _(j=j):
                            pltpu.make_async_remote_copy(
                                src_ref=y_src.at[j],
                                dst_ref=recv2d.at[pl.ds(my * m_per, m_per), :],
                                send_sem=send_sems.at[j],
                                recv_sem=recv_sems.at[my],
                                device_id=(j,),
                                device_id_type=pl.DeviceIdType.MESH,
                            ).start()

        with jax.named_scope("amax_send"):
            amax_src[0, :] = jnp.full((128,), amax, dtype=jnp.float32)
            amax_recv[my, :] = jnp.full((128,), amax, dtype=jnp.float32)
            if _DO_AMAX:
                for j in range(N_DEV):
                    @pl.when(j != my)
                    def _(j=j):
                        pltpu.make_async_remote_copy(
                            src_ref=amax_src,
                            dst_ref=amax_recv.at[pl.ds(my, 1), :],
                            send_sem=am_send_sems.at[j],
                            recv_sem=am_recv_sems.at[my],
                            device_id=(j,),
                            device_id_type=pl.DeviceIdType.MESH,
                        ).start()

        with jax.named_scope("amax_wait"):
            if _DO_AMAX:
                for s in range(N_DEV):
                    @pl.when(s != my)
                    def _(s=s):
                        pltpu.make_async_remote_copy(
                            src_ref=amax_src,
                            dst_ref=amax_recv.at[pl.ds(s, 1), :],
                            send_sem=am_send_sems.at[s],
                            recv_sem=am_recv_sems.at[s],
                            device_id=(my,),
                            device_id_type=pl.DeviceIdType.MESH,
                        ).wait_recv()
                g_amax = jnp.max(amax_recv[:, :])
            else:
                g_amax = amax
        inv_scale = 127.0 / g_amax
        scale = g_amax / 127.0

        with jax.named_scope("data_wait"):
            for s in range(N_DEV) if _DO_DATA else ():
                @pl.when(s != my)
                def _(s=s):
                    pltpu.make_async_remote_copy(
                        src_ref=y_src.at[s],
                        dst_ref=recv2d.at[pl.ds(s * m_per, m_per), :],
                        send_sem=send_sems.at[s],
                        recv_sem=recv_sems.at[s],
                        device_id=(my,),
                        device_id_type=pl.DeviceIdType.MESH,
                    ).wait_recv()

        with jax.named_scope("quant"):
            yf = recv2d[:, :].astype(jnp.float32)
            q = jnp.clip(jnp.round(yf * inv_scale), -127.0, 127.0)
            out_ref[:, :] = q * scale

        with jax.named_scope("drain"):
            for j in range(N_DEV):
                if _DO_DATA:
                    @pl.when(j != my)
                    def _(j=j):
                        pltpu.make_async_remote_copy(
                            src_ref=y_src.at[j],
                            dst_ref=recv2d.at[pl.ds(my * m_per, m_per), :],
                            send_sem=send_sems.at[j],
                            recv_sem=recv_sems.at[my],
                            device_id=(j,),
                            device_id_type=pl.DeviceIdType.MESH,
                        ).wait_send()
                if _DO_AMAX:
                    @pl.when(j != my)
                    def _(j=j):
                        pltpu.make_async_remote_copy(
                            src_ref=amax_src,
                            dst_ref=amax_recv.at[pl.ds(my, 1), :],
                            send_sem=am_send_sems.at[j],
                            recv_sem=am_recv_sems.at[my],
                            device_id=(j,),
                            device_id_type=pl.DeviceIdType.MESH,
                        ).wait_send()

    return pl.pallas_call(
        body,
        out_shape=jax.ShapeDtypeStruct((N_DEV * m_per, n_per), jnp.float32),
        in_specs=[
            pl.BlockSpec(memory_space=pltpu.VMEM),
            pl.BlockSpec(memory_space=pltpu.MemorySpace.HBM),
        ],
        out_specs=pl.BlockSpec(memory_space=pltpu.VMEM),
        scratch_shapes=[
            pltpu.VMEM((N_CHUNKS, k, n // N_CHUNKS), jnp.float32),
            pltpu.VMEM((N_DEV, m_per, n_per), jnp.bfloat16),
            pltpu.VMEM((N_DEV * m_per, n_per), jnp.bfloat16),
            pltpu.VMEM((1, 128), jnp.float32),
            pltpu.VMEM((N_DEV, 128), jnp.float32),
            pltpu.SemaphoreType.DMA((N_CHUNKS,)),
            pltpu.SemaphoreType.DMA((N_DEV,)),
            pltpu.SemaphoreType.DMA((N_DEV,)),
            pltpu.SemaphoreType.DMA((N_DEV,)),
            pltpu.SemaphoreType.DMA((N_DEV,)),
        ],
        compiler_params=pltpu.CompilerParams(
            vmem_limit_bytes=100 * 1024 * 1024,
        ),
    )(x, w_mat)
